# baseline (speedup 1.0000x reference)
import jax
import jax.numpy as jnp
from jax.experimental import pallas as pl
from jax.experimental.pallas import tpu as pltpu


def _pool_body(x_ref, o_ref):
    rb = x_ref.shape[0]
    tb = o_ref.shape[0]
    h_out, w_out = 4, 8

    lane = jax.lax.broadcasted_iota(jnp.int32, (128, w_out), 0)
    grp = jax.lax.broadcasted_iota(jnp.int32, (128, w_out), 1)
    bt = jnp.where((lane % 64) // 8 == grp, 1.0 / 128.0, 0.0).astype(
        jnp.bfloat16
    )

    y = jnp.dot(
        x_ref[...].astype(jnp.bfloat16), bt,
        preferred_element_type=jnp.float32,
    )
    z = y.reshape(tb * h_out, 8, w_out).sum(axis=1)
    z4 = z.reshape(tb, h_out, w_out)
    o_ref[...] = jnp.concatenate(
        [z4[:, h, :] for h in range(h_out)], axis=-1
    ).astype(o_ref.dtype)


@jax.jit
def _adaptive_pool(x):
    B, N, E = x.shape
    HW = 32
    x1 = x.reshape(B * N * E // 128, 128)

    tb = B
    for cand in (512, 256, 128, 64, 32, 16, 8):
        if B % cand == 0:
            tb = cand
            break
    n_blocks = B // tb
    rb = tb * (N * E // 128)

    cost = pl.CostEstimate(
        flops=2 * B * N * E * 8 + B * N * E,
        transcendentals=0,
        bytes_accessed=B * N * E * 4 + B * HW * 4,
    )
    return pl.pallas_call(
        _pool_body,
        out_shape=jax.ShapeDtypeStruct((B, HW), x.dtype),
        grid=(n_blocks,),
        in_specs=[pl.BlockSpec((rb, 128), lambda b: (b, 0))],
        out_specs=pl.BlockSpec((tb, HW), lambda b: (b, 0)),
        compiler_params=pltpu.CompilerParams(
            dimension_semantics=("arbitrary",),
        ),
        cost_estimate=cost,
    )(x1)


def kernel(x):
    return _adaptive_pool(x)


# speedup vs baseline: 9.1305x; 9.1305x over previous
import jax
import jax.numpy as jnp
from jax.experimental import pallas as pl
from jax.experimental.pallas import tpu as pltpu


def _pool_body(x_ref, o_ref):
    n, e, bb = x_ref.shape
    h_out, w_out = 4, 8
    rows_per = n // h_out
    cols_per = e // w_out
    v = x_ref[...]
    s = v.reshape(h_out, rows_per, e, bb).sum(axis=1)
    t = s.reshape(h_out, w_out, cols_per, bb).sum(axis=2)
    o_ref[...] = t * (1.0 / float(rows_per * cols_per))


@jax.jit
def _adaptive_pool(x):
    B, N, E = x.shape
    H, W = 4, 8

    xt = jnp.transpose(x, (1, 2, 0))

    bb = B
    for cand in (512, 256, 128):
        if B % cand == 0:
            bb = cand
            break
    n_blocks = B // bb

    cost = pl.CostEstimate(
        flops=B * N * E,
        transcendentals=0,
        bytes_accessed=B * N * E * 4 + B * H * W * 4,
    )
    out_t = pl.pallas_call(
        _pool_body,
        out_shape=jax.ShapeDtypeStruct((H, W, B), jnp.float32),
        grid=(n_blocks,),
        in_specs=[pl.BlockSpec((N, E, bb), lambda i: (0, 0, i))],
        out_specs=pl.BlockSpec((H, W, bb), lambda i: (0, 0, i)),
        compiler_params=pltpu.CompilerParams(
            dimension_semantics=("arbitrary",),
        ),
        cost_estimate=cost,
    )(xt)
    return jnp.transpose(out_t.reshape(H * W, B)).astype(x.dtype)


def kernel(x):
    return _adaptive_pool(x)
